# baseline (speedup 1.0000x reference)
import functools

import jax
import jax.numpy as jnp
from jax import lax
from jax.experimental import pallas as pl
from jax.experimental.pallas import tpu as pltpu
from jax.experimental.pallas import tpu_sc as plsc

N = 10000
E = 320000
D_IN = 128
H = 64
G = 64

NC = 2
NS = 16
NW = NC * NS
EPW = E // NW
B = 128
CH = 80
EPWP = CH * B
NP = 10240
NROW = NP // NS
DEGW = 16

_mesh = plsc.VectorSubcoreMesh(core_axis_name="c", subcore_axis_name="s")
_sc_params = pltpu.CompilerParams(use_tc_tiling_on_sc=False)


def _zero_vmem(ref, nrows, width):
  def row(i, _):
    for j in range(width // 16):
      ref[i, pl.ds(j * 16, 16)] = jnp.zeros((16,), jnp.float32)
    return 0
  lax.fori_loop(0, nrows, row, 0, unroll=4)


@functools.partial(
    pl.kernel,
    out_type=jax.ShapeDtypeStruct((NC, NP, DEGW), jnp.float32),
    mesh=_mesh,
    compiler_params=_sc_params,
    scratch_types=[
        pltpu.VMEM_SHARED((NP, DEGW), jnp.float32),
        pltpu.VMEM((CH, B), jnp.int32),
        pltpu.VMEM((B, DEGW), jnp.float32),
        pltpu.VMEM((B, DEGW), jnp.float32),
    ],
)
def _sc_deg(dst_hbm, out_hbm, acc_sp, idx_d, ones_v, zbuf):
  cid = lax.axis_index("c")
  sid = lax.axis_index("s")
  wid = cid * NS + sid

  _zero_vmem(zbuf, B, DEGW)
  def onesrow(i, _):
    ones_v[i, pl.ds(0, 16)] = jnp.ones((16,), jnp.float32)
    return 0
  lax.fori_loop(0, B, onesrow, 0, unroll=4)

  for z in range(NROW // B):
    pltpu.sync_copy(zbuf, acc_sp.at[pl.ds(sid * NROW + z * B, B)])
  pltpu.sync_copy(dst_hbm.at[wid], idx_d)
  plsc.subcore_barrier()

  def body(j, _):
    pltpu.sync_copy(ones_v, acc_sp.at[idx_d.at[j]], add=True)
    return 0
  lax.fori_loop(0, CH, body, 0)

  plsc.subcore_barrier()
  pltpu.sync_copy(acc_sp.at[pl.ds(sid * NROW, NROW)],
                  out_hbm.at[cid, pl.ds(sid * NROW, NROW)])


@functools.partial(
    pl.kernel,
    out_type=jax.ShapeDtypeStruct((NC, NP, H), jnp.float32),
    mesh=_mesh,
    compiler_params=_sc_params,
    scratch_types=[
        pltpu.VMEM_SHARED((NP, H), jnp.float32),
        pltpu.VMEM((CH, B), jnp.int32),
        pltpu.VMEM((CH, B), jnp.int32),
        pltpu.VMEM((B, H), jnp.float32),
        pltpu.VMEM((B, H), jnp.float32),
        pltpu.VMEM((B, H), jnp.float32),
        pltpu.SemaphoreType.DMA,
        pltpu.SemaphoreType.DMA,
    ],
)
def _sc_scatter(hp_hbm, src_hbm, dst_hbm, out_hbm,
                acc_sp, idx_s, idx_d, rows_a, rows_b, zbuf, sem_a, sem_b):
  cid = lax.axis_index("c")
  sid = lax.axis_index("s")
  wid = cid * NS + sid

  _zero_vmem(zbuf, B, H)
  for z in range(NROW // B):
    pltpu.sync_copy(zbuf, acc_sp.at[pl.ds(sid * NROW + z * B, B)])
  pltpu.sync_copy(src_hbm.at[wid], idx_s)
  pltpu.sync_copy(dst_hbm.at[wid], idx_d)
  plsc.subcore_barrier()

  pltpu.async_copy(hp_hbm.at[idx_s.at[0]], rows_a, sem_a)
  def body(i, _):
    j0 = 2 * i
    j1 = j0 + 1
    pltpu.make_async_copy(hp_hbm.at[idx_s.at[j0]], rows_a, sem_a).wait()
    pltpu.async_copy(hp_hbm.at[idx_s.at[j1]], rows_b, sem_b)
    pltpu.sync_copy(rows_a, acc_sp.at[idx_d.at[j0]], add=True)
    pltpu.make_async_copy(hp_hbm.at[idx_s.at[j1]], rows_b, sem_b).wait()
    jn = lax.rem(j1 + 1, CH)
    pltpu.async_copy(hp_hbm.at[idx_s.at[jn]], rows_a, sem_a)
    pltpu.sync_copy(rows_b, acc_sp.at[idx_d.at[j1]], add=True)
    return 0
  lax.fori_loop(0, CH // 2, body, 0)
  pltpu.make_async_copy(hp_hbm.at[idx_s.at[0]], rows_a, sem_a).wait()

  plsc.subcore_barrier()
  pltpu.sync_copy(acc_sp.at[pl.ds(sid * NROW, NROW)],
                  out_hbm.at[cid, pl.ds(sid * NROW, NROW)])


def _tc_first_body(deg2_ref, x_ref, w_ref, dinv_ref, hp_ref):
  deg = deg2_ref[0, 0:N, 0:1] + deg2_ref[1, 0:N, 0:1] + 1.0
  dinv = 1.0 / jnp.sqrt(deg)
  dinv_ref[...] = dinv
  hw = jnp.dot(x_ref[...], w_ref[...], preferred_element_type=jnp.float32)
  hp_ref[0:N, :] = hw * dinv
  hp_ref[N:NP, :] = jnp.zeros((NP - N, H), jnp.float32)


def _tc_first(deg2, x, w0):
  return pl.pallas_call(
      _tc_first_body,
      out_shape=[
          jax.ShapeDtypeStruct((N, 1), jnp.float32),
          jax.ShapeDtypeStruct((NP, H), jnp.float32),
      ],
  )(deg2, x, w0)


def _tc_layer_body(has_resid, has_next, *refs):
  refs = list(refs)
  acc_ref = refs.pop(0)
  hp_ref = refs.pop(0)
  dinv_ref = refs.pop(0)
  b_ref = refs.pop(0)
  g_ref = refs.pop(0)
  be_ref = refs.pop(0)
  resid_ref = refs.pop(0) if has_resid else None
  wn_ref = refs.pop(0) if has_next else None
  h_ref = refs.pop(0)
  hpn_ref = refs.pop(0) if has_next else None

  dinv = dinv_ref[...]
  conv = dinv * (hp_ref[0:N, :] + acc_ref[0, 0:N, :] + acc_ref[1, 0:N, :])
  conv = conv + b_ref[...]
  m = jnp.mean(conv, axis=0, keepdims=True)
  d = conv - m
  v = jnp.mean(d * d, axis=0, keepdims=True)
  y = g_ref[...] * d / jnp.sqrt(v + 1e-5) + be_ref[...]
  y = jnp.maximum(y, 0.0)
  if has_resid:
    y = y + resid_ref[...]
  h_ref[...] = y
  if has_next:
    hpn_ref[0:N, :] = jnp.dot(y, wn_ref[...],
                              preferred_element_type=jnp.float32) * dinv
    hpn_ref[N:NP, :] = jnp.zeros((NP - N, H), jnp.float32)


def _tc_layer(acc, hp, dinv, b, g, be, resid=None, w_next=None):
  args = [acc, hp, dinv, b.reshape(1, H), g.reshape(1, H), be.reshape(1, H)]
  if resid is not None:
    args.append(resid)
  if w_next is not None:
    args.append(w_next)
  out_shape = [jax.ShapeDtypeStruct((N, H), jnp.float32)]
  if w_next is not None:
    out_shape.append(jax.ShapeDtypeStruct((NP, H), jnp.float32))
  res = pl.pallas_call(
      functools.partial(_tc_layer_body, resid is not None, w_next is not None),
      out_shape=out_shape,
  )(*args)
  return res if w_next is not None else (res[0], None)


def _tc_head_body(h_ref, batch_ref, wm1_ref, bm1_ref, wm2_ref, bm2_ref,
                  out_ref):
  ids = lax.broadcasted_iota(jnp.int32, (N, G), 1)
  oh = (batch_ref[...] == ids).astype(jnp.float32)
  dn = (((0,), (0,)), ((), ()))
  s = lax.dot_general(oh, h_ref[...], dn, precision=lax.Precision.HIGHEST,
                      preferred_element_type=jnp.float32)
  cnt = lax.dot_general(oh, jnp.ones((N, 1), jnp.float32), dn,
                        precision=lax.Precision.HIGHEST,
                        preferred_element_type=jnp.float32)
  pooled = s / jnp.maximum(cnt, 1.0) + s
  z = jnp.dot(pooled, wm1_ref[...], preferred_element_type=jnp.float32)
  z = jnp.maximum(z + bm1_ref[...], 0.0)
  out_ref[...] = jnp.dot(z, wm2_ref[...],
                         preferred_element_type=jnp.float32) + bm2_ref[...]


def _tc_head(h, batch2, wm1, bm1, wm2, bm2):
  return pl.pallas_call(
      _tc_head_body,
      out_shape=jax.ShapeDtypeStruct((G, 1), jnp.float32),
  )(h, batch2, wm1, bm1.reshape(1, H // 2), wm2, bm2.reshape(1, 1))


@jax.jit
def kernel(x, edge_index, batch, W0, b0, g0, be0, W1, b1, g1, be1,
           W2, b2, g2, be2, Wm1, bm1, Wm2, bm2):
  pad = jnp.full((NW, EPWP - EPW), NP - 1, dtype=jnp.int32)
  src_r = jnp.concatenate(
      [edge_index[0].reshape(NW, EPW), pad], axis=1).reshape(NW, CH, B)
  dst_r = jnp.concatenate(
      [edge_index[1].reshape(NW, EPW), pad], axis=1).reshape(NW, CH, B)
  batch2 = batch.reshape(N, 1)

  deg2 = _sc_deg(dst_r)
  dinv, hp = _tc_first(deg2, x, W0)

  acc = _sc_scatter(hp, src_r, dst_r)
  h, hp = _tc_layer(acc, hp, dinv, b0, g0, be0, resid=None, w_next=W1)

  acc = _sc_scatter(hp, src_r, dst_r)
  h, hp = _tc_layer(acc, hp, dinv, b1, g1, be1, resid=h, w_next=W2)

  acc = _sc_scatter(hp, src_r, dst_r)
  h, _ = _tc_layer(acc, hp, dinv, b2, g2, be2, resid=h, w_next=None)

  return _tc_head(h, batch2, Wm1, bm1, Wm2, bm2)


# speedup vs baseline: 2.0911x; 2.0911x over previous
import functools

import jax
import jax.numpy as jnp
from jax import lax
from jax.experimental import pallas as pl
from jax.experimental.pallas import tpu as pltpu
from jax.experimental.pallas import tpu_sc as plsc

N = 10000
E = 320000
D_IN = 128
H = 64
G = 64

NC = 2
NS = 16
NW = NC * NS
EPW = E // NW
B = 128
CH = 80
EPWP = CH * B
NP = 10240
NROW = NP // NS
DEGW = 16

_mesh = plsc.VectorSubcoreMesh(core_axis_name="c", subcore_axis_name="s")
_sc_params = pltpu.CompilerParams(use_tc_tiling_on_sc=False)


def _zero_vmem(ref, nrows, width):
  def row(i, _):
    for j in range(width // 16):
      ref[i, pl.ds(j * 16, 16)] = jnp.zeros((16,), jnp.float32)
    return 0
  lax.fori_loop(0, nrows, row, 0, unroll=4)


@functools.partial(
    pl.kernel,
    out_type=jax.ShapeDtypeStruct((NC, NP, DEGW), jnp.float32),
    mesh=_mesh,
    compiler_params=_sc_params,
    scratch_types=[
        pltpu.VMEM_SHARED((NP, DEGW), jnp.float32),
        pltpu.VMEM((CH, B), jnp.int32),
        pltpu.VMEM((B, DEGW), jnp.float32),
        pltpu.VMEM((B, DEGW), jnp.float32),
    ],
)
def _sc_deg(dst_hbm, out_hbm, acc_sp, idx_d, ones_v, zbuf):
  cid = lax.axis_index("c")
  sid = lax.axis_index("s")
  wid = cid * NS + sid

  _zero_vmem(zbuf, B, DEGW)
  def onesrow(i, _):
    ones_v[i, pl.ds(0, 16)] = jnp.ones((16,), jnp.float32)
    return 0
  lax.fori_loop(0, B, onesrow, 0, unroll=4)

  for z in range(NROW // B):
    pltpu.sync_copy(zbuf, acc_sp.at[pl.ds(sid * NROW + z * B, B)])
  pltpu.sync_copy(dst_hbm.at[wid], idx_d)
  plsc.subcore_barrier()

  def body(j, _):
    pltpu.sync_copy(ones_v, acc_sp.at[idx_d.at[j]], add=True)
    return 0
  lax.fori_loop(0, CH, body, 0)

  plsc.subcore_barrier()
  pltpu.sync_copy(acc_sp.at[pl.ds(sid * NROW, NROW)],
                  out_hbm.at[cid, pl.ds(sid * NROW, NROW)])


@functools.partial(
    pl.kernel,
    out_type=jax.ShapeDtypeStruct((NC, NP, H), jnp.float32),
    mesh=_mesh,
    compiler_params=_sc_params,
    scratch_types=[
        pltpu.VMEM_SHARED((NP, H), jnp.float32),
        pltpu.VMEM_SHARED((NP, H), jnp.float32),
        pltpu.VMEM((CH, B), jnp.int32),
        pltpu.VMEM((CH, B), jnp.int32),
        pltpu.VMEM((B, H), jnp.float32),
        pltpu.VMEM((B, H), jnp.float32),
        pltpu.SemaphoreType.DMA,
        pltpu.SemaphoreType.DMA,
    ],
)
def _sc_scatter(hp_hbm, src_hbm, dst_hbm, out_hbm,
                hp_sp, acc_sp, idx_s, idx_d, rows_a, rows_b, sem_a, sem_b):
  cid = lax.axis_index("c")
  sid = lax.axis_index("s")
  wid = cid * NS + sid

  _zero_vmem(rows_a, B, H)
  for z in range(NROW // B):
    pltpu.sync_copy(rows_a, acc_sp.at[pl.ds(sid * NROW + z * B, B)])
  pltpu.sync_copy(hp_hbm.at[pl.ds(sid * NROW, NROW)],
                  hp_sp.at[pl.ds(sid * NROW, NROW)])
  pltpu.sync_copy(src_hbm.at[wid], idx_s)
  pltpu.sync_copy(dst_hbm.at[wid], idx_d)
  plsc.subcore_barrier()

  pltpu.async_copy(hp_sp.at[idx_s.at[0]], rows_a, sem_a)
  def body(i, _):
    j0 = 2 * i
    j1 = j0 + 1
    pltpu.make_async_copy(hp_sp.at[idx_s.at[j0]], rows_a, sem_a).wait()
    pltpu.async_copy(hp_sp.at[idx_s.at[j1]], rows_b, sem_b)
    pltpu.sync_copy(rows_a, acc_sp.at[idx_d.at[j0]], add=True)
    pltpu.make_async_copy(hp_sp.at[idx_s.at[j1]], rows_b, sem_b).wait()
    jn = lax.rem(j1 + 1, CH)
    pltpu.async_copy(hp_sp.at[idx_s.at[jn]], rows_a, sem_a)
    pltpu.sync_copy(rows_b, acc_sp.at[idx_d.at[j1]], add=True)
    return 0
  lax.fori_loop(0, CH // 2, body, 0)
  pltpu.make_async_copy(hp_sp.at[idx_s.at[0]], rows_a, sem_a).wait()

  plsc.subcore_barrier()
  pltpu.sync_copy(acc_sp.at[pl.ds(sid * NROW, NROW)],
                  out_hbm.at[cid, pl.ds(sid * NROW, NROW)])


def _tc_first_body(deg2_ref, x_ref, w_ref, dinv_ref, hp_ref):
  deg = deg2_ref[0, 0:N, 0:1] + deg2_ref[1, 0:N, 0:1] + 1.0
  dinv = 1.0 / jnp.sqrt(deg)
  dinv_ref[...] = dinv
  hw = jnp.dot(x_ref[...], w_ref[...], preferred_element_type=jnp.float32)
  hp_ref[0:N, :] = hw * dinv
  hp_ref[N:NP, :] = jnp.zeros((NP - N, H), jnp.float32)


def _tc_first(deg2, x, w0):
  return pl.pallas_call(
      _tc_first_body,
      out_shape=[
          jax.ShapeDtypeStruct((N, 1), jnp.float32),
          jax.ShapeDtypeStruct((NP, H), jnp.float32),
      ],
  )(deg2, x, w0)


def _tc_layer_body(has_resid, has_next, *refs):
  refs = list(refs)
  acc_ref = refs.pop(0)
  hp_ref = refs.pop(0)
  dinv_ref = refs.pop(0)
  b_ref = refs.pop(0)
  g_ref = refs.pop(0)
  be_ref = refs.pop(0)
  resid_ref = refs.pop(0) if has_resid else None
  wn_ref = refs.pop(0) if has_next else None
  h_ref = refs.pop(0)
  hpn_ref = refs.pop(0) if has_next else None

  dinv = dinv_ref[...]
  conv = dinv * (hp_ref[0:N, :] + acc_ref[0, 0:N, :] + acc_ref[1, 0:N, :])
  conv = conv + b_ref[...]
  m = jnp.mean(conv, axis=0, keepdims=True)
  d = conv - m
  v = jnp.mean(d * d, axis=0, keepdims=True)
  y = g_ref[...] * d / jnp.sqrt(v + 1e-5) + be_ref[...]
  y = jnp.maximum(y, 0.0)
  if has_resid:
    y = y + resid_ref[...]
  h_ref[...] = y
  if has_next:
    hpn_ref[0:N, :] = jnp.dot(y, wn_ref[...],
                              preferred_element_type=jnp.float32) * dinv
    hpn_ref[N:NP, :] = jnp.zeros((NP - N, H), jnp.float32)


def _tc_layer(acc, hp, dinv, b, g, be, resid=None, w_next=None):
  args = [acc, hp, dinv, b.reshape(1, H), g.reshape(1, H), be.reshape(1, H)]
  if resid is not None:
    args.append(resid)
  if w_next is not None:
    args.append(w_next)
  out_shape = [jax.ShapeDtypeStruct((N, H), jnp.float32)]
  if w_next is not None:
    out_shape.append(jax.ShapeDtypeStruct((NP, H), jnp.float32))
  res = pl.pallas_call(
      functools.partial(_tc_layer_body, resid is not None, w_next is not None),
      out_shape=out_shape,
  )(*args)
  return res if w_next is not None else (res[0], None)


def _tc_head_body(h_ref, batch_ref, wm1_ref, bm1_ref, wm2_ref, bm2_ref,
                  out_ref):
  ids = lax.broadcasted_iota(jnp.int32, (N, G), 1)
  oh = (batch_ref[...] == ids).astype(jnp.float32)
  dn = (((0,), (0,)), ((), ()))
  s = lax.dot_general(oh, h_ref[...], dn, precision=lax.Precision.HIGHEST,
                      preferred_element_type=jnp.float32)
  cnt = lax.dot_general(oh, jnp.ones((N, 1), jnp.float32), dn,
                        precision=lax.Precision.HIGHEST,
                        preferred_element_type=jnp.float32)
  pooled = s / jnp.maximum(cnt, 1.0) + s
  z = jnp.dot(pooled, wm1_ref[...], preferred_element_type=jnp.float32)
  z = jnp.maximum(z + bm1_ref[...], 0.0)
  out_ref[...] = jnp.dot(z, wm2_ref[...],
                         preferred_element_type=jnp.float32) + bm2_ref[...]


def _tc_head(h, batch2, wm1, bm1, wm2, bm2):
  return pl.pallas_call(
      _tc_head_body,
      out_shape=jax.ShapeDtypeStruct((G, 1), jnp.float32),
  )(h, batch2, wm1, bm1.reshape(1, H // 2), wm2, bm2.reshape(1, 1))


@jax.jit
def kernel(x, edge_index, batch, W0, b0, g0, be0, W1, b1, g1, be1,
           W2, b2, g2, be2, Wm1, bm1, Wm2, bm2):
  pad = jnp.full((NW, EPWP - EPW), NP - 1, dtype=jnp.int32)
  src_r = jnp.concatenate(
      [edge_index[0].reshape(NW, EPW), pad], axis=1).reshape(NW, CH, B)
  dst_r = jnp.concatenate(
      [edge_index[1].reshape(NW, EPW), pad], axis=1).reshape(NW, CH, B)
  batch2 = batch.reshape(N, 1)

  deg2 = _sc_deg(dst_r)
  dinv, hp = _tc_first(deg2, x, W0)

  acc = _sc_scatter(hp, src_r, dst_r)
  h, hp = _tc_layer(acc, hp, dinv, b0, g0, be0, resid=None, w_next=W1)

  acc = _sc_scatter(hp, src_r, dst_r)
  h, hp = _tc_layer(acc, hp, dinv, b1, g1, be1, resid=h, w_next=W2)

  acc = _sc_scatter(hp, src_r, dst_r)
  h, _ = _tc_layer(acc, hp, dinv, b2, g2, be2, resid=h, w_next=None)

  return _tc_head(h, batch2, Wm1, bm1, Wm2, bm2)


# speedup vs baseline: 2.4167x; 1.1557x over previous
import functools

import jax
import jax.numpy as jnp
from jax import lax
from jax.experimental import pallas as pl
from jax.experimental.pallas import tpu as pltpu
from jax.experimental.pallas import tpu_sc as plsc

N = 10000
E = 320000
D_IN = 128
H = 64
G = 64

NC = 2
NS = 16
NW = NC * NS
EPW = E // NW
B = 112
CH = 90
EPWP = CH * B
NP = 10240
NROW = NP // NS
DEGW = 16

_mesh = plsc.VectorSubcoreMesh(core_axis_name="c", subcore_axis_name="s")
_sc_params = pltpu.CompilerParams(use_tc_tiling_on_sc=False)


def _zero_vmem(ref, nrows, width):
  def row(i, _):
    for j in range(width // 16):
      ref[i, pl.ds(j * 16, 16)] = jnp.zeros((16,), jnp.float32)
    return 0
  lax.fori_loop(0, nrows, row, 0, unroll=4)


@functools.partial(
    pl.kernel,
    out_type=jax.ShapeDtypeStruct((NC, NP, DEGW), jnp.float32),
    mesh=_mesh,
    compiler_params=_sc_params,
    scratch_types=[
        pltpu.VMEM_SHARED((NP, DEGW), jnp.float32),
        pltpu.VMEM((CH, B), jnp.int32),
        pltpu.VMEM((B, DEGW), jnp.float32),
        pltpu.VMEM((B, DEGW), jnp.float32),
    ],
)
def _sc_deg(dst_hbm, out_hbm, acc_sp, idx_d, ones_v, zbuf):
  cid = lax.axis_index("c")
  sid = lax.axis_index("s")
  wid = cid * NS + sid

  _zero_vmem(zbuf, B, DEGW)
  def onesrow(i, _):
    ones_v[i, pl.ds(0, 16)] = jnp.ones((16,), jnp.float32)
    return 0
  lax.fori_loop(0, B, onesrow, 0, unroll=4)

  for z in range(NROW // 80):
    pltpu.sync_copy(zbuf.at[pl.ds(0, 80)],
                    acc_sp.at[pl.ds(sid * NROW + z * 80, 80)])
  pltpu.sync_copy(dst_hbm.at[wid], idx_d)
  plsc.subcore_barrier()

  def body(j, _):
    pltpu.sync_copy(ones_v, acc_sp.at[idx_d.at[j]], add=True)
    return 0
  lax.fori_loop(0, CH, body, 0)

  plsc.subcore_barrier()
  pltpu.sync_copy(acc_sp.at[pl.ds(sid * NROW, NROW)],
                  out_hbm.at[cid, pl.ds(sid * NROW, NROW)])


@functools.partial(
    pl.kernel,
    out_type=jax.ShapeDtypeStruct((NC, NP, H), jnp.float32),
    mesh=_mesh,
    compiler_params=_sc_params,
    scratch_types=[
        pltpu.VMEM_SHARED((NP, H), jnp.float32),
        pltpu.VMEM_SHARED((NP, H), jnp.float32),
        pltpu.VMEM((CH, B), jnp.int32),
        pltpu.VMEM((CH, B), jnp.int32),
        pltpu.VMEM((B, H), jnp.float32),
        pltpu.VMEM((B, H), jnp.float32),
        pltpu.VMEM((B, H), jnp.float32),
        pltpu.VMEM((B, H), jnp.float32),
        [pltpu.SemaphoreType.DMA] * 4,
        [pltpu.SemaphoreType.DMA] * 4,
    ],
)
def _sc_scatter(hp_hbm, src_hbm, dst_hbm, out_hbm,
                hp_sp, acc_sp, idx_s, idx_d, r0, r1, r2, r3, gsem, ssem):
  cid = lax.axis_index("c")
  sid = lax.axis_index("s")
  wid = cid * NS + sid
  rows = [r0, r1, r2, r3]

  _zero_vmem(r0, B, H)
  for z in range(NROW // 80):
    pltpu.sync_copy(r0.at[pl.ds(0, 80)],
                    acc_sp.at[pl.ds(sid * NROW + z * 80, 80)])
  pltpu.sync_copy(hp_hbm.at[pl.ds(sid * NROW, NROW)],
                  hp_sp.at[pl.ds(sid * NROW, NROW)])
  pltpu.sync_copy(src_hbm.at[wid], idx_s)
  pltpu.sync_copy(dst_hbm.at[wid], idx_d)
  plsc.subcore_barrier()

  def gather(j, k):
    pltpu.async_copy(hp_sp.at[idx_s.at[j]], rows[k], gsem[k])

  def gwait(j, k):
    pltpu.make_async_copy(hp_sp.at[idx_s.at[j]], rows[k], gsem[k]).wait()

  def scat(j, k):
    pltpu.async_copy(rows[k], acc_sp.at[idx_d.at[j]], ssem[k], add=True)

  def swait(j, k):
    pltpu.make_async_copy(rows[k], acc_sp.at[idx_d.at[j]], ssem[k]).wait()

  gather(0, 0)
  gather(1, 1)
  gwait(0, 0); scat(0, 0); gather(2, 2)
  gwait(1, 1); scat(1, 1); gather(3, 3)
  def body(i, _):
    for k in range(4):
      j = 4 * i + 2 + k
      kb = (2 + k) % 4
      kf = k
      gwait(j, kb)
      scat(j, kb)
      swait(j - 2, kf)
      jn = lax.rem(j + 2, CH)
      gather(jn, kf)
    return 0
  lax.fori_loop(0, (CH - 2) // 4, body, 0)
  swait(CH - 2, (CH - 2) % 4)
  swait(CH - 1, (CH - 1) % 4)
  gwait(0, CH % 4)
  gwait(1, (CH + 1) % 4)

  plsc.subcore_barrier()
  pltpu.sync_copy(acc_sp.at[pl.ds(sid * NROW, NROW)],
                  out_hbm.at[cid, pl.ds(sid * NROW, NROW)])


def _tc_first_body(deg2_ref, x_ref, w_ref, dinv_ref, hp_ref):
  deg = deg2_ref[0, 0:N, 0:1] + deg2_ref[1, 0:N, 0:1] + 1.0
  dinv = 1.0 / jnp.sqrt(deg)
  dinv_ref[...] = dinv
  hw = jnp.dot(x_ref[...], w_ref[...], preferred_element_type=jnp.float32)
  hp_ref[0:N, :] = hw * dinv
  hp_ref[N:NP, :] = jnp.zeros((NP - N, H), jnp.float32)


def _tc_first(deg2, x, w0):
  return pl.pallas_call(
      _tc_first_body,
      out_shape=[
          jax.ShapeDtypeStruct((N, 1), jnp.float32),
          jax.ShapeDtypeStruct((NP, H), jnp.float32),
      ],
  )(deg2, x, w0)


def _tc_layer_body(has_resid, has_next, *refs):
  refs = list(refs)
  acc_ref = refs.pop(0)
  hp_ref = refs.pop(0)
  dinv_ref = refs.pop(0)
  b_ref = refs.pop(0)
  g_ref = refs.pop(0)
  be_ref = refs.pop(0)
  resid_ref = refs.pop(0) if has_resid else None
  wn_ref = refs.pop(0) if has_next else None
  h_ref = refs.pop(0)
  hpn_ref = refs.pop(0) if has_next else None

  dinv = dinv_ref[...]
  conv = dinv * (hp_ref[0:N, :] + acc_ref[0, 0:N, :] + acc_ref[1, 0:N, :])
  conv = conv + b_ref[...]
  m = jnp.mean(conv, axis=0, keepdims=True)
  d = conv - m
  v = jnp.mean(d * d, axis=0, keepdims=True)
  y = g_ref[...] * d / jnp.sqrt(v + 1e-5) + be_ref[...]
  y = jnp.maximum(y, 0.0)
  if has_resid:
    y = y + resid_ref[...]
  h_ref[...] = y
  if has_next:
    hpn_ref[0:N, :] = jnp.dot(y, wn_ref[...],
                              preferred_element_type=jnp.float32) * dinv
    hpn_ref[N:NP, :] = jnp.zeros((NP - N, H), jnp.float32)


def _tc_layer(acc, hp, dinv, b, g, be, resid=None, w_next=None):
  args = [acc, hp, dinv, b.reshape(1, H), g.reshape(1, H), be.reshape(1, H)]
  if resid is not None:
    args.append(resid)
  if w_next is not None:
    args.append(w_next)
  out_shape = [jax.ShapeDtypeStruct((N, H), jnp.float32)]
  if w_next is not None:
    out_shape.append(jax.ShapeDtypeStruct((NP, H), jnp.float32))
  res = pl.pallas_call(
      functools.partial(_tc_layer_body, resid is not None, w_next is not None),
      out_shape=out_shape,
  )(*args)
  return res if w_next is not None else (res[0], None)


def _tc_head_body(acc_ref, hp_ref, dinv_ref, b_ref, g_ref, be_ref,
                  resid_ref, batch_ref, wm1_ref, bm1_ref, wm2_ref, bm2_ref,
                  out_ref):
  dinv = dinv_ref[...]
  conv = dinv * (hp_ref[0:N, :] + acc_ref[0, 0:N, :] + acc_ref[1, 0:N, :])
  conv = conv + b_ref[...]
  m = jnp.mean(conv, axis=0, keepdims=True)
  d = conv - m
  v = jnp.mean(d * d, axis=0, keepdims=True)
  y = g_ref[...] * d / jnp.sqrt(v + 1e-5) + be_ref[...]
  h = jnp.maximum(y, 0.0) + resid_ref[...]

  ids = lax.broadcasted_iota(jnp.int32, (N, G), 1)
  oh = (batch_ref[...] == ids).astype(jnp.float32)
  dn = (((0,), (0,)), ((), ()))
  s = lax.dot_general(oh, h, dn, precision=lax.Precision.HIGHEST,
                      preferred_element_type=jnp.float32)
  cnt = lax.dot_general(oh, jnp.ones((N, 1), jnp.float32), dn,
                        precision=lax.Precision.HIGHEST,
                        preferred_element_type=jnp.float32)
  pooled = s / jnp.maximum(cnt, 1.0) + s
  z = jnp.dot(pooled, wm1_ref[...], preferred_element_type=jnp.float32)
  z = jnp.maximum(z + bm1_ref[...], 0.0)
  out_ref[...] = jnp.dot(z, wm2_ref[...],
                         preferred_element_type=jnp.float32) + bm2_ref[...]


def _tc_head(acc, hp, dinv, b, g, be, resid, batch2, wm1, bm1, wm2, bm2):
  return pl.pallas_call(
      _tc_head_body,
      out_shape=jax.ShapeDtypeStruct((G, 1), jnp.float32),
  )(acc, hp, dinv, b.reshape(1, H), g.reshape(1, H), be.reshape(1, H),
    resid, batch2, wm1, bm1.reshape(1, H // 2), wm2, bm2.reshape(1, 1))


@jax.jit
def kernel(x, edge_index, batch, W0, b0, g0, be0, W1, b1, g1, be1,
           W2, b2, g2, be2, Wm1, bm1, Wm2, bm2):
  pad = jnp.full((NW, EPWP - EPW), NP - 1, dtype=jnp.int32)
  src_r = jnp.concatenate(
      [edge_index[0].reshape(NW, EPW), pad], axis=1).reshape(NW, CH, B)
  dst_r = jnp.concatenate(
      [edge_index[1].reshape(NW, EPW), pad], axis=1).reshape(NW, CH, B)
  batch2 = batch.reshape(N, 1)

  deg2 = _sc_deg(dst_r)
  dinv, hp = _tc_first(deg2, x, W0)

  acc = _sc_scatter(hp, src_r, dst_r)
  h, hp = _tc_layer(acc, hp, dinv, b0, g0, be0, resid=None, w_next=W1)

  acc = _sc_scatter(hp, src_r, dst_r)
  h, hp = _tc_layer(acc, hp, dinv, b1, g1, be1, resid=h, w_next=W2)

  acc = _sc_scatter(hp, src_r, dst_r)
  return _tc_head(acc, hp, dinv, b2, g2, be2, h, batch2, Wm1, bm1, Wm2, bm2)


# speedup vs baseline: 2.4168x; 1.0000x over previous
import functools

import jax
import jax.numpy as jnp
from jax import lax
from jax.experimental import pallas as pl
from jax.experimental.pallas import tpu as pltpu
from jax.experimental.pallas import tpu_sc as plsc

N = 10000
E = 320000
D_IN = 128
H = 64
G = 64

NC = 2
NS = 16
NW = NC * NS
EPW = E // NW
B = 112
CH = 90
EPWP = CH * B
NP = 10240
NROW = NP // NS
DEGW = 16

_mesh = plsc.VectorSubcoreMesh(core_axis_name="c", subcore_axis_name="s")
_sc_params = pltpu.CompilerParams(use_tc_tiling_on_sc=False)


def _zero_vmem(ref, nrows, width):
  def row(i, _):
    for j in range(width // 16):
      ref[i, pl.ds(j * 16, 16)] = jnp.zeros((16,), jnp.float32)
    return 0
  lax.fori_loop(0, nrows, row, 0, unroll=4)


@functools.partial(
    pl.kernel,
    out_type=jax.ShapeDtypeStruct((NC, NP, DEGW), jnp.float32),
    mesh=_mesh,
    compiler_params=_sc_params,
    scratch_types=[
        pltpu.VMEM_SHARED((NP, DEGW), jnp.float32),
        pltpu.VMEM((CH, B), jnp.int32),
        pltpu.VMEM((B, DEGW), jnp.float32),
        pltpu.VMEM((B, DEGW), jnp.float32),
    ],
)
def _sc_deg(dst_hbm, out_hbm, acc_sp, idx_d, ones_v, zbuf):
  cid = lax.axis_index("c")
  sid = lax.axis_index("s")
  wid = cid * NS + sid

  _zero_vmem(zbuf, B, DEGW)
  def onesrow(i, _):
    ones_v[i, pl.ds(0, 16)] = jnp.ones((16,), jnp.float32)
    return 0
  lax.fori_loop(0, B, onesrow, 0, unroll=4)

  for z in range(NROW // 80):
    pltpu.sync_copy(zbuf.at[pl.ds(0, 80)],
                    acc_sp.at[pl.ds(sid * NROW + z * 80, 80)])
  pltpu.sync_copy(dst_hbm.at[wid], idx_d)
  plsc.subcore_barrier()

  def body(j, _):
    pltpu.sync_copy(ones_v, acc_sp.at[idx_d.at[j]], add=True)
    return 0
  lax.fori_loop(0, CH, body, 0)

  plsc.subcore_barrier()
  pltpu.sync_copy(acc_sp.at[pl.ds(sid * NROW, NROW)],
                  out_hbm.at[cid, pl.ds(sid * NROW, NROW)])


@functools.partial(
    pl.kernel,
    out_type=jax.ShapeDtypeStruct((NC, NP, H), jnp.float32),
    mesh=_mesh,
    compiler_params=_sc_params,
    scratch_types=[
        pltpu.VMEM_SHARED((NP, H), jnp.float32),
        pltpu.VMEM_SHARED((NP, H), jnp.float32),
        pltpu.VMEM((CH, B), jnp.int32),
        pltpu.VMEM((CH, B), jnp.int32),
        pltpu.VMEM((B, H), jnp.float32),
        pltpu.VMEM((B, H), jnp.float32),
        pltpu.VMEM((B, H), jnp.float32),
        pltpu.VMEM((B, H), jnp.float32),
        [pltpu.SemaphoreType.DMA] * 4,
        [pltpu.SemaphoreType.DMA] * 4,
    ],
)
def _sc_scatter(hp_hbm, src_hbm, dst_hbm, out_hbm,
                hp_sp, acc_sp, idx_s, idx_d, r0, r1, r2, r3, gsem, ssem):
  cid = lax.axis_index("c")
  sid = lax.axis_index("s")
  wid = cid * NS + sid
  rows = [r0, r1, r2, r3]

  _zero_vmem(r0, B, H)
  for z in range(NROW // 80):
    pltpu.sync_copy(r0.at[pl.ds(0, 80)],
                    acc_sp.at[pl.ds(sid * NROW + z * 80, 80)])
  pltpu.sync_copy(hp_hbm.at[pl.ds(sid * NROW, NROW)],
                  hp_sp.at[pl.ds(sid * NROW, NROW)])
  pltpu.sync_copy(src_hbm.at[wid], idx_s)
  pltpu.sync_copy(dst_hbm.at[wid], idx_d)
  plsc.subcore_barrier()

  def gather(j, k):
    pltpu.async_copy(hp_sp.at[idx_s.at[j]], rows[k], gsem[k])

  def gwait(j, k):
    pltpu.make_async_copy(hp_sp.at[idx_s.at[j]], rows[k], gsem[k]).wait()

  def scat(j, k):
    pltpu.async_copy(rows[k], acc_sp.at[idx_d.at[j]], ssem[k], add=True)

  def swait(j, k):
    pltpu.make_async_copy(rows[k], acc_sp.at[idx_d.at[j]], ssem[k]).wait()

  gather(0, 0)
  gather(1, 1)
  gwait(0, 0); scat(0, 0); gather(2, 2)
  gwait(1, 1); scat(1, 1); gather(3, 3)
  def body(i, _):
    for k in range(4):
      j = 4 * i + 2 + k
      kb = (2 + k) % 4
      kf = k
      gwait(j, kb)
      scat(j, kb)
      swait(j - 2, kf)
      jn = lax.rem(j + 2, CH)
      gather(jn, kf)
    return 0
  lax.fori_loop(0, (CH - 2) // 4, body, 0)
  swait(CH - 2, (CH - 2) % 4)
  swait(CH - 1, (CH - 1) % 4)
  gwait(0, CH % 4)
  gwait(1, (CH + 1) % 4)

  plsc.subcore_barrier()
  pltpu.sync_copy(acc_sp.at[pl.ds(sid * NROW, NROW)],
                  out_hbm.at[cid, pl.ds(sid * NROW, NROW)])


def _tc_mm_body(x_ref, w_ref, hw_ref):
  hw_ref[...] = jnp.dot(x_ref[...], w_ref[...],
                        preferred_element_type=jnp.float32)


def _tc_mm(x, w0):
  return pl.pallas_call(
      _tc_mm_body,
      out_shape=jax.ShapeDtypeStruct((N, H), jnp.float32),
  )(x, w0)


def _tc_first_body(deg2_ref, hw_ref, dinv_ref, hp_ref):
  deg = deg2_ref[0, 0:N, 0:1] + deg2_ref[1, 0:N, 0:1] + 1.0
  dinv = 1.0 / jnp.sqrt(deg)
  dinv_ref[...] = dinv
  hp_ref[0:N, :] = hw_ref[...] * dinv
  hp_ref[N:NP, :] = jnp.zeros((NP - N, H), jnp.float32)


def _tc_first(deg2, hw):
  return pl.pallas_call(
      _tc_first_body,
      out_shape=[
          jax.ShapeDtypeStruct((N, 1), jnp.float32),
          jax.ShapeDtypeStruct((NP, H), jnp.float32),
      ],
  )(deg2, hw)


def _tc_layer_body(has_resid, has_next, *refs):
  refs = list(refs)
  acc_ref = refs.pop(0)
  hp_ref = refs.pop(0)
  dinv_ref = refs.pop(0)
  b_ref = refs.pop(0)
  g_ref = refs.pop(0)
  be_ref = refs.pop(0)
  resid_ref = refs.pop(0) if has_resid else None
  wn_ref = refs.pop(0) if has_next else None
  h_ref = refs.pop(0)
  hpn_ref = refs.pop(0) if has_next else None

  dinv = dinv_ref[...]
  conv = dinv * (hp_ref[0:N, :] + acc_ref[0, 0:N, :] + acc_ref[1, 0:N, :])
  conv = conv + b_ref[...]
  m = jnp.mean(conv, axis=0, keepdims=True)
  d = conv - m
  v = jnp.mean(d * d, axis=0, keepdims=True)
  y = g_ref[...] * d / jnp.sqrt(v + 1e-5) + be_ref[...]
  y = jnp.maximum(y, 0.0)
  if has_resid:
    y = y + resid_ref[...]
  h_ref[...] = y
  if has_next:
    hpn_ref[0:N, :] = jnp.dot(y, wn_ref[...],
                              preferred_element_type=jnp.float32) * dinv
    hpn_ref[N:NP, :] = jnp.zeros((NP - N, H), jnp.float32)


def _tc_layer(acc, hp, dinv, b, g, be, resid=None, w_next=None):
  args = [acc, hp, dinv, b.reshape(1, H), g.reshape(1, H), be.reshape(1, H)]
  if resid is not None:
    args.append(resid)
  if w_next is not None:
    args.append(w_next)
  out_shape = [jax.ShapeDtypeStruct((N, H), jnp.float32)]
  if w_next is not None:
    out_shape.append(jax.ShapeDtypeStruct((NP, H), jnp.float32))
  res = pl.pallas_call(
      functools.partial(_tc_layer_body, resid is not None, w_next is not None),
      out_shape=out_shape,
  )(*args)
  return res if w_next is not None else (res[0], None)


def _tc_head_body(acc_ref, hp_ref, dinv_ref, b_ref, g_ref, be_ref,
                  resid_ref, batch_ref, wm1_ref, bm1_ref, wm2_ref, bm2_ref,
                  out_ref):
  dinv = dinv_ref[...]
  conv = dinv * (hp_ref[0:N, :] + acc_ref[0, 0:N, :] + acc_ref[1, 0:N, :])
  conv = conv + b_ref[...]
  m = jnp.mean(conv, axis=0, keepdims=True)
  d = conv - m
  v = jnp.mean(d * d, axis=0, keepdims=True)
  y = g_ref[...] * d / jnp.sqrt(v + 1e-5) + be_ref[...]
  h = jnp.maximum(y, 0.0) + resid_ref[...]

  ids = lax.broadcasted_iota(jnp.int32, (N, G), 1)
  oh = (batch_ref[...] == ids).astype(jnp.float32)
  dn = (((0,), (0,)), ((), ()))
  s = lax.dot_general(oh, h, dn, precision=lax.Precision.HIGHEST,
                      preferred_element_type=jnp.float32)
  cnt = lax.dot_general(oh, jnp.ones((N, 1), jnp.float32), dn,
                        precision=lax.Precision.HIGHEST,
                        preferred_element_type=jnp.float32)
  pooled = s / jnp.maximum(cnt, 1.0) + s
  z = jnp.dot(pooled, wm1_ref[...], preferred_element_type=jnp.float32)
  z = jnp.maximum(z + bm1_ref[...], 0.0)
  out_ref[...] = jnp.dot(z, wm2_ref[...],
                         preferred_element_type=jnp.float32) + bm2_ref[...]


def _tc_head(acc, hp, dinv, b, g, be, resid, batch2, wm1, bm1, wm2, bm2):
  return pl.pallas_call(
      _tc_head_body,
      out_shape=jax.ShapeDtypeStruct((G, 1), jnp.float32),
  )(acc, hp, dinv, b.reshape(1, H), g.reshape(1, H), be.reshape(1, H),
    resid, batch2, wm1, bm1.reshape(1, H // 2), wm2, bm2.reshape(1, 1))


@jax.jit
def kernel(x, edge_index, batch, W0, b0, g0, be0, W1, b1, g1, be1,
           W2, b2, g2, be2, Wm1, bm1, Wm2, bm2):
  pad = jnp.full((NW, EPWP - EPW), NP - 1, dtype=jnp.int32)
  src_r = jnp.concatenate(
      [edge_index[0].reshape(NW, EPW), pad], axis=1).reshape(NW, CH, B)
  dst_r = jnp.concatenate(
      [edge_index[1].reshape(NW, EPW), pad], axis=1).reshape(NW, CH, B)
  batch2 = batch.reshape(N, 1)

  hw0 = _tc_mm(x, W0)
  deg2 = _sc_deg(dst_r)
  dinv, hp = _tc_first(deg2, hw0)

  acc = _sc_scatter(hp, src_r, dst_r)
  h, hp = _tc_layer(acc, hp, dinv, b0, g0, be0, resid=None, w_next=W1)

  acc = _sc_scatter(hp, src_r, dst_r)
  h, hp = _tc_layer(acc, hp, dinv, b1, g1, be1, resid=h, w_next=W2)

  acc = _sc_scatter(hp, src_r, dst_r)
  return _tc_head(acc, hp, dinv, b2, g2, be2, h, batch2, Wm1, bm1, Wm2, bm2)
